# Initial kernel scaffold; baseline (speedup 1.0000x reference)
import jax
import jax.numpy as jnp
from jax.experimental import pallas as pl


def kernel(x, idx0, w0, idx1, w1, idx2, w2, Wout):
    raise NotImplementedError("write your pallas kernel here")


# speedup vs baseline: 1.6774x; 1.6774x over previous
import jax
import jax.numpy as jnp
from jax.experimental import pallas as pl

B = 1024
F = 128
BITS = 8
ENC = F * BITS
H = 2048
K = 32
C = 1000

_HB = 256
_BB = 256


def _build_wt_kernel(idx_ref, w_ref, wt_ref):
    hb, kk = idx_ref.shape
    prev = wt_ref.shape[1]
    iota = jax.lax.broadcasted_iota(jnp.int32, (hb, prev), 1)
    acc = jnp.zeros((hb, prev), jnp.float32)
    for k in range(kk):
        acc = acc + jnp.where(idx_ref[:, k:k + 1] == iota,
                              w_ref[:, k:k + 1], 0.0)
    wt_ref[...] = acc


def _build_wt(idx, w, prev):
    return pl.pallas_call(
        _build_wt_kernel,
        grid=(H // _HB,),
        in_specs=[
            pl.BlockSpec((_HB, K), lambda i: (i, 0)),
            pl.BlockSpec((_HB, K), lambda i: (i, 0)),
        ],
        out_specs=pl.BlockSpec((_HB, prev), lambda i: (i, 0)),
        out_shape=jax.ShapeDtypeStruct((H, prev), jnp.float32),
    )(idx, w)


def _layer0_kernel(x_ref, wt_ref, wout_ref, act_ref, out_ref):
    x = x_ref[...]
    bb = x.shape[0]
    j = jax.lax.broadcasted_iota(jnp.int32, (bb, F, BITS), 2)
    th = (j.astype(jnp.float32) + 0.5) * (1.0 / BITS)
    code = (x[:, :, None] > th).astype(jnp.float32)
    code = code.reshape(bb, ENC)
    z = jax.lax.dot_general(code, wt_ref[...], (((1,), (1,)), ((), ())),
                            preferred_element_type=jnp.float32)
    a = (z > 0.0).astype(jnp.float32)
    act_ref[...] = a
    out_ref[...] = jnp.dot(a, wout_ref[...],
                           preferred_element_type=jnp.float32)


def _layer_kernel(act_in_ref, wt_ref, wout_ref, out_in_ref, act_ref, out_ref):
    z = jax.lax.dot_general(act_in_ref[...], wt_ref[...],
                            (((1,), (1,)), ((), ())),
                            preferred_element_type=jnp.float32)
    a = (z > 0.0).astype(jnp.float32)
    act_ref[...] = a
    out_ref[...] = out_in_ref[...] + jnp.dot(a, wout_ref[...],
                                             preferred_element_type=jnp.float32)


def _layer0(x, wt0, wout0):
    return pl.pallas_call(
        _layer0_kernel,
        grid=(B // _BB,),
        in_specs=[
            pl.BlockSpec((_BB, F), lambda i: (i, 0)),
            pl.BlockSpec((H, ENC), lambda i: (0, 0)),
            pl.BlockSpec((H, C), lambda i: (0, 0)),
        ],
        out_specs=[
            pl.BlockSpec((_BB, H), lambda i: (i, 0)),
            pl.BlockSpec((_BB, C), lambda i: (i, 0)),
        ],
        out_shape=[
            jax.ShapeDtypeStruct((B, H), jnp.float32),
            jax.ShapeDtypeStruct((B, C), jnp.float32),
        ],
    )(x, wt0, wout0)


def _layer(act, wt, wout, out_in):
    return pl.pallas_call(
        _layer_kernel,
        grid=(B // _BB,),
        in_specs=[
            pl.BlockSpec((_BB, H), lambda i: (i, 0)),
            pl.BlockSpec((H, H), lambda i: (0, 0)),
            pl.BlockSpec((H, C), lambda i: (0, 0)),
            pl.BlockSpec((_BB, C), lambda i: (i, 0)),
        ],
        out_specs=[
            pl.BlockSpec((_BB, H), lambda i: (i, 0)),
            pl.BlockSpec((_BB, C), lambda i: (i, 0)),
        ],
        out_shape=[
            jax.ShapeDtypeStruct((B, H), jnp.float32),
            jax.ShapeDtypeStruct((B, C), jnp.float32),
        ],
    )(act, wt, wout, out_in)


def kernel(x, idx0, w0, idx1, w1, idx2, w2, Wout):
    wt0 = _build_wt(idx0, w0, ENC)
    wt1 = _build_wt(idx1, w1, H)
    wt2 = _build_wt(idx2, w2, H)
    act1, out0 = _layer0(x, wt0, Wout[0])
    act2, out1 = _layer(act1, wt1, Wout[1], out0)
    _, out2 = _layer(act2, wt2, Wout[2], out1)
    return out2


# speedup vs baseline: 2.0258x; 1.2078x over previous
import jax
import jax.numpy as jnp
from jax.experimental import pallas as pl
from jax.experimental.pallas import tpu as pltpu

B = 1024
F = 128
BITS = 8
ENC = F * BITS
H = 2048
K = 32
C = 1000

_HB = 256
_BB = 256


def _build_wt_kernel(idx_ref, w_ref, wt_ref, *, permute):
    hb, kk = idx_ref.shape
    prev = wt_ref.shape[1]
    idx = idx_ref[...]
    if permute:
        idx = (idx % BITS) * F + idx // BITS
    iota = jax.lax.broadcasted_iota(jnp.int32, (hb, prev), 1)
    acc = jnp.zeros((hb, prev), jnp.float32)
    for k in range(kk):
        acc = acc + jnp.where(idx[:, k:k + 1] == iota,
                              w_ref[:, k:k + 1], 0.0)
    wt_ref[...] = acc.astype(jnp.bfloat16)


def _build_wt(idx, w, prev, permute=False):
    import functools
    return pl.pallas_call(
        functools.partial(_build_wt_kernel, permute=permute),
        grid=(H // _HB,),
        in_specs=[
            pl.BlockSpec((_HB, K), lambda i: (i, 0)),
            pl.BlockSpec((_HB, K), lambda i: (i, 0)),
        ],
        out_specs=pl.BlockSpec((_HB, prev), lambda i: (i, 0)),
        out_shape=jax.ShapeDtypeStruct((H, prev), jnp.bfloat16),
        compiler_params=pltpu.CompilerParams(
            dimension_semantics=("parallel",)),
    )(idx, w)


def _layer0_kernel(x_ref, wt_ref, wout_ref, act_ref, out_ref):
    x = x_ref[...]
    code = jnp.concatenate(
        [(x > (j + 0.5) * (1.0 / BITS)).astype(jnp.bfloat16)
         for j in range(BITS)], axis=1)
    z = jax.lax.dot_general(code, wt_ref[...], (((1,), (1,)), ((), ())),
                            preferred_element_type=jnp.float32)
    a = (z > 0.0).astype(jnp.bfloat16)
    act_ref[...] = a
    out_ref[...] = jnp.dot(a, wout_ref[...],
                           preferred_element_type=jnp.float32)


def _layer_kernel(act_in_ref, wt_ref, wout_ref, out_in_ref, act_ref, out_ref):
    z = jax.lax.dot_general(act_in_ref[...], wt_ref[...],
                            (((1,), (1,)), ((), ())),
                            preferred_element_type=jnp.float32)
    a = (z > 0.0).astype(jnp.bfloat16)
    act_ref[...] = a
    out_ref[...] = out_in_ref[...] + jnp.dot(a, wout_ref[...],
                                             preferred_element_type=jnp.float32)


def _layer0(x, wt0, wout0):
    return pl.pallas_call(
        _layer0_kernel,
        grid=(B // _BB,),
        in_specs=[
            pl.BlockSpec((_BB, F), lambda i: (i, 0)),
            pl.BlockSpec((H, ENC), lambda i: (0, 0)),
            pl.BlockSpec((H, C), lambda i: (0, 0)),
        ],
        out_specs=[
            pl.BlockSpec((_BB, H), lambda i: (i, 0)),
            pl.BlockSpec((_BB, C), lambda i: (i, 0)),
        ],
        out_shape=[
            jax.ShapeDtypeStruct((B, H), jnp.bfloat16),
            jax.ShapeDtypeStruct((B, C), jnp.float32),
        ],
        compiler_params=pltpu.CompilerParams(
            dimension_semantics=("parallel",)),
    )(x, wt0, wout0)


def _layer(act, wt, wout, out_in):
    return pl.pallas_call(
        _layer_kernel,
        grid=(B // _BB,),
        in_specs=[
            pl.BlockSpec((_BB, H), lambda i: (i, 0)),
            pl.BlockSpec((H, H), lambda i: (0, 0)),
            pl.BlockSpec((H, C), lambda i: (0, 0)),
            pl.BlockSpec((_BB, C), lambda i: (i, 0)),
        ],
        out_specs=[
            pl.BlockSpec((_BB, H), lambda i: (i, 0)),
            pl.BlockSpec((_BB, C), lambda i: (i, 0)),
        ],
        out_shape=[
            jax.ShapeDtypeStruct((B, H), jnp.bfloat16),
            jax.ShapeDtypeStruct((B, C), jnp.float32),
        ],
        compiler_params=pltpu.CompilerParams(
            dimension_semantics=("parallel",)),
    )(act, wt, wout, out_in)


def kernel(x, idx0, w0, idx1, w1, idx2, w2, Wout):
    wt0 = _build_wt(idx0, w0, ENC, permute=True)
    wt1 = _build_wt(idx1, w1, H)
    wt2 = _build_wt(idx2, w2, H)
    wout = Wout.astype(jnp.bfloat16)
    act1, out0 = _layer0(x, wt0, wout[0])
    act2, out1 = _layer(act1, wt1, wout[1], out0)
    _, out2 = _layer(act2, wt2, wout[2], out1)
    return out2


# speedup vs baseline: 2.7379x; 1.3515x over previous
import functools

import jax
import jax.numpy as jnp
from jax import lax
from jax.experimental import pallas as pl
from jax.experimental.pallas import tpu as pltpu
from jax.experimental.pallas import tpu_sc as plsc

B = 1024
F = 128
BITS = 8
ENC = F * BITS
H = 2048
K = 32
C = 1000

NC = 2
NS = 16
NW = NC * NS
ROWS_W = H // NW
BUFW = ROWS_W * ENC

_BB = 512

_SC_MESH = plsc.VectorSubcoreMesh(
    core_axis_name="c", subcore_axis_name="s",
    num_cores=NC, num_subcores=NS)


def _sc_build_kernel(idx0_ref, w0_ref, idx1_ref, w1_ref, idx2_ref, w2_ref,
                     wt0_ref, wt1_ref, wt2_ref, buf, idx_v, w_v):
    wid = lax.axis_index("s") * NC + lax.axis_index("c")

    def zero_body(i, _):
        base = pl.multiple_of(i * 128, 128)
        for j in range(8):
            buf[pl.ds(base + j * 16, 16)] = jnp.zeros((16,), jnp.float32)
        return 0

    lax.fori_loop(0, BUFW // 128, zero_body, 0)

    def scatter(n, prev, permute, sign):
        kper = K // 16

        def body(u, _):
            iv = idx_v[pl.ds(u * 16, 16)]
            if permute:
                iv = (iv & (BITS - 1)) * F + (iv >> 3)
            fv = iv + (u // kper) * prev
            wv = w_v[pl.ds(u * 16, 16)]
            plsc.addupdate_scatter(buf, [fv], wv * sign)
            return 0

        lax.fori_loop(0, n // 16, body, 0)

    def chunk(idx_hbm, w_hbm, wt_hbm, prev, rows, cid, permute, restore):
        n = rows * K
        off = wid * ROWS_W * K + cid * n
        pltpu.sync_copy(idx_hbm.at[pl.ds(off, n)], idx_v.at[pl.ds(0, n)])
        pltpu.sync_copy(w_hbm.at[pl.ds(off, n)], w_v.at[pl.ds(0, n)])
        scatter(n, prev, permute, 1.0)
        out_off = (wid * ROWS_W + cid * rows) * prev
        pltpu.sync_copy(buf.at[pl.ds(0, rows * prev)],
                        wt_hbm.at[pl.ds(out_off, rows * prev)])
        if restore:
            scatter(n, prev, permute, -1.0)

    chunk(idx0_ref, w0_ref, wt0_ref, ENC, ROWS_W, 0, True, True)
    chunk(idx1_ref, w1_ref, wt1_ref, H, ROWS_W // 2, 0, False, True)
    chunk(idx1_ref, w1_ref, wt1_ref, H, ROWS_W // 2, 1, False, True)
    chunk(idx2_ref, w2_ref, wt2_ref, H, ROWS_W // 2, 0, False, True)
    chunk(idx2_ref, w2_ref, wt2_ref, H, ROWS_W // 2, 1, False, False)


_sc_build = pl.kernel(
    _sc_build_kernel,
    out_type=[
        jax.ShapeDtypeStruct((H * ENC,), jnp.float32),
        jax.ShapeDtypeStruct((H * H,), jnp.float32),
        jax.ShapeDtypeStruct((H * H,), jnp.float32),
    ],
    mesh=_SC_MESH,
    scratch_types=[
        pltpu.VMEM((BUFW,), jnp.float32),
        pltpu.VMEM((ROWS_W * K,), jnp.int32),
        pltpu.VMEM((ROWS_W * K,), jnp.float32),
    ],
    compiler_params=pltpu.CompilerParams(needs_layout_passes=False),
)


def _cast_kernel(src_ref, dst_ref):
    dst_ref[...] = src_ref[...].astype(jnp.bfloat16)


def _cast_bf16(wt_flat, prev):
    hb = 256
    return pl.pallas_call(
        _cast_kernel,
        grid=(H // hb,),
        in_specs=[pl.BlockSpec((hb, prev), lambda i: (i, 0))],
        out_specs=pl.BlockSpec((hb, prev), lambda i: (i, 0)),
        out_shape=jax.ShapeDtypeStruct((H, prev), jnp.bfloat16),
        compiler_params=pltpu.CompilerParams(
            dimension_semantics=("parallel",)),
    )(wt_flat.reshape(H, prev))


def _fwd_kernel(x_ref, wt0_ref, wt1_ref, wt2_ref,
                wo0_ref, wo1_ref, wo2_ref, out_ref):
    x = x_ref[...]
    code = jnp.concatenate(
        [(x > (j + 0.5) * (1.0 / BITS)).astype(jnp.bfloat16)
         for j in range(BITS)], axis=1)
    dn = (((1,), (1,)), ((), ()))
    z0 = lax.dot_general(code, wt0_ref[...], dn,
                         preferred_element_type=jnp.float32)
    a1 = (z0 > 0.0).astype(jnp.bfloat16)
    z1 = lax.dot_general(a1, wt1_ref[...], dn,
                         preferred_element_type=jnp.float32)
    a2 = (z1 > 0.0).astype(jnp.bfloat16)
    z2 = lax.dot_general(a2, wt2_ref[...], dn,
                         preferred_element_type=jnp.float32)
    a3 = (z2 > 0.0).astype(jnp.bfloat16)
    out = jnp.dot(a1, wo0_ref[...], preferred_element_type=jnp.float32)
    out = out + jnp.dot(a2, wo1_ref[...], preferred_element_type=jnp.float32)
    out = out + jnp.dot(a3, wo2_ref[...], preferred_element_type=jnp.float32)
    out_ref[...] = out


def _forward(x, wt0, wt1, wt2, wout):
    return pl.pallas_call(
        _fwd_kernel,
        grid=(B // _BB,),
        in_specs=[
            pl.BlockSpec((_BB, F), lambda i: (i, 0)),
            pl.BlockSpec((H, ENC), lambda i: (0, 0)),
            pl.BlockSpec((H, H), lambda i: (0, 0)),
            pl.BlockSpec((H, H), lambda i: (0, 0)),
            pl.BlockSpec((H, C), lambda i: (0, 0)),
            pl.BlockSpec((H, C), lambda i: (0, 0)),
            pl.BlockSpec((H, C), lambda i: (0, 0)),
        ],
        out_specs=pl.BlockSpec((_BB, C), lambda i: (i, 0)),
        out_shape=jax.ShapeDtypeStruct((B, C), jnp.float32),
        compiler_params=pltpu.CompilerParams(
            dimension_semantics=("parallel",)),
    )(x, wt0, wt1, wt2, wout[0], wout[1], wout[2])


def kernel(x, idx0, w0, idx1, w1, idx2, w2, Wout):
    wt0f, wt1f, wt2f = _sc_build(
        idx0.reshape(-1).astype(jnp.int32), w0.reshape(-1),
        idx1.reshape(-1).astype(jnp.int32), w1.reshape(-1),
        idx2.reshape(-1).astype(jnp.int32), w2.reshape(-1))
    wt0 = _cast_bf16(wt0f, ENC)
    wt1 = _cast_bf16(wt1f, H)
    wt2 = _cast_bf16(wt2f, H)
    return _forward(x, wt0, wt1, wt2, Wout.astype(jnp.bfloat16))


# speedup vs baseline: 2.9855x; 1.0904x over previous
import jax
import jax.numpy as jnp
from jax import lax
from jax.experimental import pallas as pl
from jax.experimental.pallas import tpu as pltpu
from jax.experimental.pallas import tpu_sc as plsc

B = 1024
F = 128
BITS = 8
ENC = F * BITS
H = 2048
K = 32
C = 1000

NC = 2
NS = 16
NW = NC * NS
ROWS_W = H // NW
BUFW = ROWS_W * ENC

_BB = 512

_SC_MESH = plsc.VectorSubcoreMesh(
    core_axis_name="c", subcore_axis_name="s",
    num_cores=NC, num_subcores=NS)


def _sc_build_body(idx_ref, w_ref, wt_ref, buf, idx_v, w_v, *, prev, permute):
    wid = lax.axis_index("s") * NC + lax.axis_index("c")

    def zero_body(i, _):
        base = pl.multiple_of(i * 128, 128)
        for j in range(8):
            buf[pl.ds(base + j * 16, 16)] = jnp.zeros((16,), jnp.float32)
        return 0

    lax.fori_loop(0, BUFW // 128, zero_body, 0)

    def scatter(n, sign):
        kper = K // 16

        def body(u, _):
            iv = idx_v[pl.ds(u * 16, 16)]
            if permute:
                iv = (iv & (BITS - 1)) * F + (iv >> 3)
            fv = iv + (u // kper) * prev
            wv = w_v[pl.ds(u * 16, 16)]
            plsc.addupdate_scatter(buf, [fv], wv * sign)
            return 0

        lax.fori_loop(0, n // 16, body, 0)

    rows = BUFW // prev
    nchunks = ROWS_W // rows
    for cid in range(nchunks):
        n = rows * K
        off = wid * ROWS_W * K + cid * n
        pltpu.sync_copy(idx_ref.at[pl.ds(off, n)], idx_v.at[pl.ds(0, n)])
        pltpu.sync_copy(w_ref.at[pl.ds(off, n)], w_v.at[pl.ds(0, n)])
        scatter(n, 1.0)
        out_off = (wid * ROWS_W + cid * rows) * prev
        pltpu.sync_copy(buf, wt_ref.at[pl.ds(out_off, rows * prev)])
        if cid + 1 < nchunks:
            scatter(n, -1.0)


def _sc_build(idx, w, prev, permute):
    import functools
    body = functools.partial(_sc_build_body, prev=prev, permute=permute)
    built = pl.kernel(
        body,
        out_type=jax.ShapeDtypeStruct((H * prev,), jnp.float32),
        mesh=_SC_MESH,
        scratch_types=[
            pltpu.VMEM((BUFW,), jnp.float32),
            pltpu.VMEM((ROWS_W * K,), jnp.int32),
            pltpu.VMEM((ROWS_W * K,), jnp.float32),
        ],
        compiler_params=pltpu.CompilerParams(needs_layout_passes=False),
    )(idx.reshape(-1).astype(jnp.int32), w.reshape(-1))
    return built.reshape(H, prev)


def _layer0_kernel(x_ref, wt_ref, wout_ref, act_ref, out_ref):
    x = x_ref[...]
    code = jnp.concatenate(
        [(x > (j + 0.5) * (1.0 / BITS)).astype(jnp.bfloat16)
         for j in range(BITS)], axis=1)
    wt = wt_ref[...].astype(jnp.bfloat16)
    z = lax.dot_general(code, wt, (((1,), (1,)), ((), ())),
                        preferred_element_type=jnp.float32)
    a = (z > 0.0).astype(jnp.bfloat16)
    act_ref[...] = a
    out_ref[...] = jnp.dot(a, wout_ref[...],
                           preferred_element_type=jnp.float32)


def _layer_kernel(act_in_ref, wt_ref, wout_ref, out_in_ref, act_ref, out_ref):
    wt = wt_ref[...].astype(jnp.bfloat16)
    z = lax.dot_general(act_in_ref[...], wt, (((1,), (1,)), ((), ())),
                        preferred_element_type=jnp.float32)
    a = (z > 0.0).astype(jnp.bfloat16)
    act_ref[...] = a
    out_ref[...] = out_in_ref[...] + jnp.dot(a, wout_ref[...],
                                             preferred_element_type=jnp.float32)


def _layer0(x, wt0, wout0):
    return pl.pallas_call(
        _layer0_kernel,
        grid=(B // _BB,),
        in_specs=[
            pl.BlockSpec((_BB, F), lambda i: (i, 0)),
            pl.BlockSpec((H, ENC), lambda i: (0, 0)),
            pl.BlockSpec((H, C), lambda i: (0, 0)),
        ],
        out_specs=[
            pl.BlockSpec((_BB, H), lambda i: (i, 0)),
            pl.BlockSpec((_BB, C), lambda i: (i, 0)),
        ],
        out_shape=[
            jax.ShapeDtypeStruct((B, H), jnp.bfloat16),
            jax.ShapeDtypeStruct((B, C), jnp.float32),
        ],
        compiler_params=pltpu.CompilerParams(
            dimension_semantics=("parallel",)),
    )(x, wt0, wout0)


def _layer(act, wt, wout, out_in):
    return pl.pallas_call(
        _layer_kernel,
        grid=(B // _BB,),
        in_specs=[
            pl.BlockSpec((_BB, H), lambda i: (i, 0)),
            pl.BlockSpec((H, H), lambda i: (0, 0)),
            pl.BlockSpec((H, C), lambda i: (0, 0)),
            pl.BlockSpec((_BB, C), lambda i: (i, 0)),
        ],
        out_specs=[
            pl.BlockSpec((_BB, H), lambda i: (i, 0)),
            pl.BlockSpec((_BB, C), lambda i: (i, 0)),
        ],
        out_shape=[
            jax.ShapeDtypeStruct((B, H), jnp.bfloat16),
            jax.ShapeDtypeStruct((B, C), jnp.float32),
        ],
        compiler_params=pltpu.CompilerParams(
            dimension_semantics=("parallel",)),
    )(act, wt, wout, out_in)


def kernel(x, idx0, w0, idx1, w1, idx2, w2, Wout):
    wt0 = _sc_build(idx0, w0, ENC, True)
    wt1 = _sc_build(idx1, w1, H, False)
    wt2 = _sc_build(idx2, w2, H, False)
    wout = Wout.astype(jnp.bfloat16)
    act1, out0 = _layer0(x, wt0, wout[0])
    act2, out1 = _layer(act1, wt1, wout[1], out0)
    _, out2 = _layer(act2, wt2, wout[2], out1)
    return out2


# speedup vs baseline: 4.1010x; 1.3736x over previous
import functools

import jax
import jax.numpy as jnp
from jax import lax
from jax.experimental import pallas as pl
from jax.experimental.pallas import tpu as pltpu
from jax.experimental.pallas import tpu_sc as plsc

B = 1024
F = 128
BITS = 8
ENC = F * BITS
H = 2048
K = 32
C = 1000

NC = 2
NS = 16
NW = NC * NS
ROWS_W = H // NW
BUFW = ROWS_W * ENC

_BB = 512

_SC_MESH = plsc.VectorSubcoreMesh(
    core_axis_name="c", subcore_axis_name="s",
    num_cores=NC, num_subcores=NS)


def _sc_build_body(idx_ref, w_ref, wt_ref, buf, idx_v, w_v, *, prev, permute):
    wid = lax.axis_index("s") * NC + lax.axis_index("c")
    rows = BUFW // prev
    nchunks = ROWS_W // rows
    vec_per_row = prev // 16
    kper = K // 16

    def zero_body(i, _):
        r = i // (vec_per_row // 8)
        base = (i % (vec_per_row // 8)) * 128
        for j in range(8):
            buf[r, pl.ds(base + j * 16, 16)] = jnp.zeros((16,), jnp.float32)
        return 0

    lax.fori_loop(0, rows * (vec_per_row // 8), zero_body, 0)

    def scatter(sign):
        def body(u, _):
            r = u // kper
            iv = idx_v[r, pl.ds((u % kper) * 16, 16)]
            if permute:
                iv = (iv & (BITS - 1)) * F + (iv >> 3)
            wv = w_v[r, pl.ds((u % kper) * 16, 16)]
            rv = jnp.full((16,), r, jnp.int32)
            plsc.addupdate_scatter(buf, [rv, iv], wv * sign)
            return 0

        lax.fori_loop(0, rows * kper, body, 0)

    for cid in range(nchunks):
        row0 = wid * ROWS_W + cid * rows
        pltpu.sync_copy(idx_ref.at[pl.ds(row0, rows)], idx_v.at[pl.ds(0, rows)])
        pltpu.sync_copy(w_ref.at[pl.ds(row0, rows)], w_v.at[pl.ds(0, rows)])
        scatter(1.0)
        pltpu.sync_copy(buf, wt_ref.at[pl.ds(row0, rows)])
        if cid + 1 < nchunks:
            scatter(-1.0)


def _sc_build(idx, w, prev, permute):
    rows = BUFW // prev
    body = functools.partial(_sc_build_body, prev=prev, permute=permute)
    return pl.kernel(
        body,
        out_type=jax.ShapeDtypeStruct((H, prev), jnp.float32),
        mesh=_SC_MESH,
        scratch_types=[
            pltpu.VMEM((rows, prev), jnp.float32),
            pltpu.VMEM((ROWS_W, K), jnp.int32),
            pltpu.VMEM((ROWS_W, K), jnp.float32),
        ],
        compiler_params=pltpu.CompilerParams(needs_layout_passes=False),
    )(idx, w)


def _layer0_kernel(x_ref, wt_ref, wout_ref, act_ref, out_ref):
    x = x_ref[...]
    code = jnp.concatenate(
        [(x > (j + 0.5) * (1.0 / BITS)).astype(jnp.bfloat16)
         for j in range(BITS)], axis=1)
    wt = wt_ref[...].astype(jnp.bfloat16)
    z = lax.dot_general(code, wt, (((1,), (1,)), ((), ())),
                        preferred_element_type=jnp.float32)
    a = (z > 0.0).astype(jnp.bfloat16)
    act_ref[...] = a
    wo = wout_ref[0].astype(jnp.bfloat16)
    out_ref[...] = jnp.dot(a, wo, preferred_element_type=jnp.float32)


def _layer_kernel(act_in_ref, wt_ref, wout_ref, out_in_ref, act_ref, out_ref):
    wt = wt_ref[...].astype(jnp.bfloat16)
    z = lax.dot_general(act_in_ref[...], wt, (((1,), (1,)), ((), ())),
                        preferred_element_type=jnp.float32)
    a = (z > 0.0).astype(jnp.bfloat16)
    act_ref[...] = a
    wo = wout_ref[0].astype(jnp.bfloat16)
    out_ref[...] = out_in_ref[...] + jnp.dot(
        a, wo, preferred_element_type=jnp.float32)


def _layer0(x, wt0, wout):
    return pl.pallas_call(
        _layer0_kernel,
        grid=(B // _BB,),
        in_specs=[
            pl.BlockSpec((_BB, F), lambda i: (i, 0)),
            pl.BlockSpec((H, ENC), lambda i: (0, 0)),
            pl.BlockSpec((1, H, C), lambda i: (0, 0, 0)),
        ],
        out_specs=[
            pl.BlockSpec((_BB, H), lambda i: (i, 0)),
            pl.BlockSpec((_BB, C), lambda i: (i, 0)),
        ],
        out_shape=[
            jax.ShapeDtypeStruct((B, H), jnp.bfloat16),
            jax.ShapeDtypeStruct((B, C), jnp.float32),
        ],
        compiler_params=pltpu.CompilerParams(
            dimension_semantics=("parallel",)),
    )(x, wt0, wout)


def _layer(act, wt, wout, lidx, out_in):
    return pl.pallas_call(
        _layer_kernel,
        grid=(B // _BB,),
        in_specs=[
            pl.BlockSpec((_BB, H), lambda i: (i, 0)),
            pl.BlockSpec((H, H), lambda i: (0, 0)),
            pl.BlockSpec((1, H, C), lambda i, l=lidx: (l, 0, 0)),
            pl.BlockSpec((_BB, C), lambda i: (i, 0)),
        ],
        out_specs=[
            pl.BlockSpec((_BB, H), lambda i: (i, 0)),
            pl.BlockSpec((_BB, C), lambda i: (i, 0)),
        ],
        out_shape=[
            jax.ShapeDtypeStruct((B, H), jnp.bfloat16),
            jax.ShapeDtypeStruct((B, C), jnp.float32),
        ],
        compiler_params=pltpu.CompilerParams(
            dimension_semantics=("parallel",)),
    )(act, wt, wout, out_in)


def kernel(x, idx0, w0, idx1, w1, idx2, w2, Wout):
    wt0 = _sc_build(idx0.astype(jnp.int32), w0, ENC, True)
    wt1 = _sc_build(idx1.astype(jnp.int32), w1, H, False)
    wt2 = _sc_build(idx2.astype(jnp.int32), w2, H, False)
    act1, out0 = _layer0(x, wt0, Wout)
    act2, out1 = _layer(act1, wt1, Wout, 1, out0)
    _, out2 = _layer(act2, wt2, Wout, 2, out1)
    return out2
